# Initial kernel scaffold; baseline (speedup 1.0000x reference)
import jax
import jax.numpy as jnp
from jax.experimental import pallas as pl


def kernel(x, gate_w, w1, w2):
    raise NotImplementedError("write your pallas kernel here")


# speedup vs baseline: 1.2706x; 1.2706x over previous
import functools

import jax
import jax.numpy as jnp
from jax import lax
from jax.experimental import pallas as pl
from jax.experimental.pallas import tpu as pltpu

HIDDEN = 1024
INTER = 4096
E = 8
TOPK = 2
LBW = 0.01


def _router_body(x_ref, g_ref, w_ref, usage_ref, lb_ref, ent_ref):
    S = x_ref.shape[0]
    logits = lax.dot_general(
        x_ref[...], g_ref[...], (((1,), (1,)), ((), ())),
        preferred_element_type=jnp.float32)
    mx = jnp.max(logits, axis=1, keepdims=True)
    ex = jnp.exp(logits - mx)
    p = ex / jnp.sum(ex, axis=1, keepdims=True)

    ent = -jnp.mean(jnp.sum(p * jnp.log(p + 1e-8), axis=1)) * 0.01
    ent_ref[...] = jnp.broadcast_to(ent, (1, 1))

    lane = lax.broadcasted_iota(jnp.int32, (S, E), 1)
    m1 = jnp.max(p, axis=1, keepdims=True)
    i1 = jnp.min(jnp.where(p == m1, lane, E), axis=1, keepdims=True)
    pm = jnp.where(lane == i1, -1.0, p)
    m2 = jnp.max(pm, axis=1, keepdims=True)
    i2 = jnp.min(jnp.where(pm == m2, lane, E), axis=1, keepdims=True)
    tot = m1 + m2
    r1 = m1 / tot
    r2 = m2 / tot
    hit1 = (lane == i1)
    hit2 = (lane == i2)
    w_ref[...] = jnp.where(hit1, r1, 0.0) + jnp.where(hit2, r2, 0.0)

    cnt = hit1.astype(jnp.float32) + hit2.astype(jnp.float32)
    usage = jnp.sum(cnt, axis=0, keepdims=True) / (S * TOPK)
    usage_ref[...] = usage
    lb = jnp.mean((usage - 1.0 / E) ** 2) * LBW
    lb_ref[...] = jnp.broadcast_to(lb, (1, 1))


def _ffn_body(x_ref, w1_ref, w2_ref, wmat_ref, out_ref):
    e = pl.program_id(0)
    c = pl.program_id(1)
    h = lax.dot_general(
        x_ref[...], w1_ref[0], (((1,), (1,)), ((), ())),
        preferred_element_type=jnp.float32)
    a = h * (1.0 / (1.0 + jnp.exp(-h)))
    y = lax.dot_general(
        a, w2_ref[0], (((1,), (1,)), ((), ())),
        preferred_element_type=jnp.float32)
    lane = lax.broadcasted_iota(jnp.int32, wmat_ref.shape, 1)
    wcol = jnp.sum(jnp.where(lane == e, wmat_ref[...], 0.0), axis=1,
                   keepdims=True)
    val = y * wcol

    @pl.when(jnp.logical_and(e == 0, c == 0))
    def _():
        out_ref[...] = val

    @pl.when(jnp.logical_or(e != 0, c != 0))
    def _():
        out_ref[...] += val


@functools.partial(jax.jit, static_argnames=())
def kernel(x, gate_w, w1, w2):
    B, S, H = x.shape
    x2 = x.reshape(S, H)

    wmat, usage, lb, ent = pl.pallas_call(
        _router_body,
        out_shape=(
            jax.ShapeDtypeStruct((S, E), jnp.float32),
            jax.ShapeDtypeStruct((1, E), jnp.float32),
            jax.ShapeDtypeStruct((1, 1), jnp.float32),
            jax.ShapeDtypeStruct((1, 1), jnp.float32),
        ),
    )(x2, gate_w)

    C = 4
    IC = INTER // C
    out = pl.pallas_call(
        _ffn_body,
        grid=(E, C),
        in_specs=[
            pl.BlockSpec((S, H), lambda e, c: (0, 0)),
            pl.BlockSpec((1, IC, H), lambda e, c: (e, c, 0)),
            pl.BlockSpec((1, H, IC), lambda e, c: (e, 0, c)),
            pl.BlockSpec((S, E), lambda e, c: (0, 0)),
        ],
        out_specs=pl.BlockSpec((S, H), lambda e, c: (0, 0)),
        out_shape=jax.ShapeDtypeStruct((S, H), jnp.float32),
    )(x2, w1, w2, wmat)

    return (out.reshape(B, S, H), lb.reshape(()), ent.reshape(()),
            usage.reshape(E))
